# Initial kernel scaffold; baseline (speedup 1.0000x reference)
import jax
import jax.numpy as jnp
from jax.experimental import pallas as pl


def kernel(logits, embedding, memory_bank, prototypes, target, dataset_ids):
    raise NotImplementedError("write your pallas kernel here")


# speedup vs baseline: 1.2886x; 1.2886x over previous
import jax
import jax.numpy as jnp
from jax.experimental import pallas as pl
from jax.experimental.pallas import tpu as pltpu

_C = 128
_P = 8
_D = 256
_STRIDE = 8
_COEF = 0.999
_LOSS_W = 0.1
_EPS = 1e-12

_SEG_BW = 2048


def _seg_ce_kernel(x_ref, t_ref, acc_ref):
    i = pl.program_id(0)
    j = pl.program_id(1)
    x = x_ref[0]
    t = t_ref[0, 0, 0]
    m = jnp.max(x, axis=0)
    s = jnp.sum(jnp.exp(x - m[None, :]), axis=0)
    lse = m + jnp.log(s)
    cls = jax.lax.broadcasted_iota(jnp.int32, x.shape, 0)
    tsel = jnp.sum(jnp.where(cls == t[None, :], x, 0.0), axis=0)
    partial = jnp.sum(lse - tsel)

    @pl.when(jnp.logical_and(i == 0, j == 0))
    def _():
        acc_ref[0, 0] = 0.0

    acc_ref[0, 0] += partial


def _bank_kernel(mb_ref, proto_ref, out_ref):
    mean = jnp.mean(mb_ref[...], axis=1)
    n1 = jnp.sqrt(jnp.sum(mean * mean, axis=-1, keepdims=True))
    nm = mean / jnp.maximum(n1, _EPS)
    blended = nm * (1.0 - _COEF) + proto_ref[...] * _COEF
    n2 = jnp.sqrt(jnp.sum(blended * blended, axis=-1, keepdims=True))
    out_ref[...] = blended / jnp.maximum(n2, _EPS)


def _contrast_kernel(emb_ref, protos_ref, lb_ref, acc_ref):
    i = pl.program_id(0)
    logits = jax.lax.dot_general(
        emb_ref[...], protos_ref[...],
        dimension_numbers=(((1,), (1,)), ((), ())),
        preferred_element_type=jnp.float32)
    lb = lb_ref[0, 0]
    m = jnp.max(logits, axis=1)
    s = jnp.sum(jnp.exp(logits - m[:, None]), axis=1)
    lse = m + jnp.log(s)
    col = jax.lax.broadcasted_iota(jnp.int32, logits.shape, 1)
    lo = (lb * _P)[:, None]
    mask = jnp.logical_and(col >= lo, col < lo + _P)
    clsmax = jnp.max(jnp.where(mask, logits, -jnp.inf), axis=1)
    partial = jnp.sum(lse - clsmax)

    @pl.when(i == 0)
    def _():
        acc_ref[0, 0] = 0.0

    acc_ref[0, 0] += partial


def kernel(logits, embedding, memory_bank, prototypes, target, dataset_ids):
    b, c, h, w = logits.shape
    hw = h * w
    n_seg = b * hw
    nb = hw // _SEG_BW

    logits_r = logits.reshape(b, c, hw)
    target_r = target.reshape(b, nb, 1, _SEG_BW)

    seg_sum = pl.pallas_call(
        _seg_ce_kernel,
        grid=(b, nb),
        in_specs=[
            pl.BlockSpec((1, c, _SEG_BW), lambda i, j: (i, 0, j)),
            pl.BlockSpec((1, 1, 1, _SEG_BW), lambda i, j: (i, j, 0, 0)),
        ],
        out_specs=pl.BlockSpec(memory_space=pltpu.SMEM),
        out_shape=jax.ShapeDtypeStruct((1, 1), jnp.float32),
    )(logits_r, target_r)

    protos = pl.pallas_call(
        _bank_kernel,
        grid=(memory_bank.shape[0] // 128,),
        in_specs=[
            pl.BlockSpec((128, memory_bank.shape[1], _D), lambda i: (i, 0, 0)),
            pl.BlockSpec((128, _D), lambda i: (i, 0)),
        ],
        out_specs=pl.BlockSpec((128, _D), lambda i: (i, 0)),
        out_shape=jax.ShapeDtypeStruct((_C * _P, _D), jnp.float32),
    )(memory_bank, prototypes)

    rearr_emb = jnp.transpose(embedding, (0, 2, 3, 1)).reshape(-1, _D)
    n_ctr = rearr_emb.shape[0]
    contrast_lb = target[:, ::_STRIDE, ::_STRIDE].reshape(-1)
    _R = 512
    nr = n_ctr // _R
    lb_r = contrast_lb.reshape(nr, 1, _R)

    ctr_sum = pl.pallas_call(
        _contrast_kernel,
        grid=(nr,),
        in_specs=[
            pl.BlockSpec((_R, _D), lambda i: (i, 0)),
            pl.BlockSpec((_C * _P, _D), lambda i: (0, 0)),
            pl.BlockSpec((1, 1, _R), lambda i: (i, 0, 0)),
        ],
        out_specs=pl.BlockSpec(memory_space=pltpu.SMEM),
        out_shape=jax.ShapeDtypeStruct((1, 1), jnp.float32),
    )(rearr_emb, protos, lb_r)

    loss_seg = seg_sum[0, 0] / n_seg
    loss_contrast = ctr_sum[0, 0] / n_ctr
    loss = loss_seg + _LOSS_W * loss_contrast
    return (loss, loss_seg, loss_contrast, protos)


# speedup vs baseline: 1.4254x; 1.1062x over previous
import jax
import jax.numpy as jnp
from jax.experimental import pallas as pl
from jax.experimental.pallas import tpu as pltpu

_C = 128
_P = 8
_D = 256
_STRIDE = 8
_COEF = 0.999
_LOSS_W = 0.1
_EPS = 1e-12

_SEG_BW = 4096


def _seg_ce_kernel(x_ref, t_ref, acc_ref):
    i = pl.program_id(0)
    j = pl.program_id(1)
    x = x_ref[0]
    t = t_ref[0, 0, 0]
    m = jnp.max(x, axis=0)
    s = jnp.sum(jnp.exp(x - m[None, :]), axis=0)
    lse = m + jnp.log(s)
    cls = jax.lax.broadcasted_iota(jnp.int32, x.shape, 0)
    tsel = jnp.sum(jnp.where(cls == t[None, :], x, 0.0), axis=0)
    partial = jnp.sum(lse - tsel)

    @pl.when(jnp.logical_and(i == 0, j == 0))
    def _():
        acc_ref[0, 0] = 0.0

    acc_ref[0, 0] += partial


def _bank_kernel(mb_ref, proto_ref, out_ref):
    mean = jnp.mean(mb_ref[...], axis=1)
    n1 = jnp.sqrt(jnp.sum(mean * mean, axis=-1, keepdims=True))
    nm = mean / jnp.maximum(n1, _EPS)
    blended = nm * (1.0 - _COEF) + proto_ref[...] * _COEF
    n2 = jnp.sqrt(jnp.sum(blended * blended, axis=-1, keepdims=True))
    out_ref[...] = blended / jnp.maximum(n2, _EPS)


def _contrast_kernel(emb_ref, protos_ref, lb_ref, acc_ref):
    i = pl.program_id(0)
    logits = jax.lax.dot_general(
        emb_ref[...], protos_ref[...],
        dimension_numbers=(((1,), (1,)), ((), ())),
        preferred_element_type=jnp.float32)
    lb = lb_ref[0, 0]
    m = jnp.max(logits, axis=1)
    s = jnp.sum(jnp.exp(logits - m[:, None]), axis=1)
    lse = m + jnp.log(s)
    col = jax.lax.broadcasted_iota(jnp.int32, logits.shape, 1)
    lo = (lb * _P)[:, None]
    mask = jnp.logical_and(col >= lo, col < lo + _P)
    clsmax = jnp.max(jnp.where(mask, logits, -jnp.inf), axis=1)
    partial = jnp.sum(lse - clsmax)

    @pl.when(i == 0)
    def _():
        acc_ref[0, 0] = 0.0

    acc_ref[0, 0] += partial


def kernel(logits, embedding, memory_bank, prototypes, target, dataset_ids):
    b, c, h, w = logits.shape
    hw = h * w
    n_seg = b * hw
    nb = hw // _SEG_BW

    logits_r = logits.reshape(b, c, hw)
    target_r = target.reshape(b, nb, 1, _SEG_BW)

    seg_sum = pl.pallas_call(
        _seg_ce_kernel,
        grid=(b, nb),
        in_specs=[
            pl.BlockSpec((1, c, _SEG_BW), lambda i, j: (i, 0, j)),
            pl.BlockSpec((1, 1, 1, _SEG_BW), lambda i, j: (i, j, 0, 0)),
        ],
        out_specs=pl.BlockSpec(memory_space=pltpu.SMEM),
        out_shape=jax.ShapeDtypeStruct((1, 1), jnp.float32),
    )(logits_r, target_r)

    protos = pl.pallas_call(
        _bank_kernel,
        grid=(memory_bank.shape[0] // 128,),
        in_specs=[
            pl.BlockSpec((128, memory_bank.shape[1], _D), lambda i: (i, 0, 0)),
            pl.BlockSpec((128, _D), lambda i: (i, 0)),
        ],
        out_specs=pl.BlockSpec((128, _D), lambda i: (i, 0)),
        out_shape=jax.ShapeDtypeStruct((_C * _P, _D), jnp.float32),
    )(memory_bank, prototypes)

    rearr_emb = jnp.transpose(embedding, (0, 2, 3, 1)).reshape(-1, _D)
    n_ctr = rearr_emb.shape[0]
    contrast_lb = target[:, ::_STRIDE, ::_STRIDE].reshape(-1)
    _R = 512
    nr = n_ctr // _R
    lb_r = contrast_lb.reshape(nr, 1, _R)

    ctr_sum = pl.pallas_call(
        _contrast_kernel,
        grid=(nr,),
        in_specs=[
            pl.BlockSpec((_R, _D), lambda i: (i, 0)),
            pl.BlockSpec((_C * _P, _D), lambda i: (0, 0)),
            pl.BlockSpec((1, 1, _R), lambda i: (i, 0, 0)),
        ],
        out_specs=pl.BlockSpec(memory_space=pltpu.SMEM),
        out_shape=jax.ShapeDtypeStruct((1, 1), jnp.float32),
    )(rearr_emb, protos, lb_r)

    loss_seg = seg_sum[0, 0] / n_seg
    loss_contrast = ctr_sum[0, 0] / n_ctr
    loss = loss_seg + _LOSS_W * loss_contrast
    return (loss, loss_seg, loss_contrast, protos)


# speedup vs baseline: 1.5004x; 1.0526x over previous
import jax
import jax.numpy as jnp
from jax.experimental import pallas as pl
from jax.experimental.pallas import tpu as pltpu

_C = 128
_P = 8
_D = 256
_STRIDE = 8
_COEF = 0.999
_LOSS_W = 0.1
_EPS = 1e-12

_SEG_BW = 16384


def _seg_ce_kernel(x_ref, t_ref, acc_ref):
    i = pl.program_id(0)
    j = pl.program_id(1)
    x = x_ref[0]
    t = t_ref[0, 0, 0]
    m = jnp.max(x, axis=0)
    s = jnp.sum(jnp.exp(x - m[None, :]), axis=0)
    lse = m + jnp.log(s)
    cls = jax.lax.broadcasted_iota(jnp.int32, x.shape, 0)
    tsel = jnp.sum(jnp.where(cls == t[None, :], x, 0.0), axis=0)
    partial = jnp.sum(lse - tsel)

    @pl.when(jnp.logical_and(i == 0, j == 0))
    def _():
        acc_ref[0, 0] = 0.0

    acc_ref[0, 0] += partial


def _bank_kernel(mb_ref, proto_ref, out_ref):
    mean = jnp.mean(mb_ref[...], axis=1)
    n1 = jnp.sqrt(jnp.sum(mean * mean, axis=-1, keepdims=True))
    nm = mean / jnp.maximum(n1, _EPS)
    blended = nm * (1.0 - _COEF) + proto_ref[...] * _COEF
    n2 = jnp.sqrt(jnp.sum(blended * blended, axis=-1, keepdims=True))
    out_ref[...] = blended / jnp.maximum(n2, _EPS)


def _contrast_kernel(emb_ref, protos_ref, lb_ref, acc_ref):
    i = pl.program_id(0)
    logits = jax.lax.dot_general(
        emb_ref[...], protos_ref[...],
        dimension_numbers=(((1,), (1,)), ((), ())),
        preferred_element_type=jnp.float32)
    lb = lb_ref[0, 0]
    m = jnp.max(logits, axis=1)
    s = jnp.sum(jnp.exp(logits - m[:, None]), axis=1)
    lse = m + jnp.log(s)
    col = jax.lax.broadcasted_iota(jnp.int32, logits.shape, 1)
    lo = (lb * _P)[:, None]
    mask = jnp.logical_and(col >= lo, col < lo + _P)
    clsmax = jnp.max(jnp.where(mask, logits, -jnp.inf), axis=1)
    partial = jnp.sum(lse - clsmax)

    @pl.when(i == 0)
    def _():
        acc_ref[0, 0] = 0.0

    acc_ref[0, 0] += partial


def kernel(logits, embedding, memory_bank, prototypes, target, dataset_ids):
    b, c, h, w = logits.shape
    hw = h * w
    n_seg = b * hw
    nb = hw // _SEG_BW

    logits_r = logits.reshape(b, c, hw)
    target_r = target.reshape(b, nb, 1, _SEG_BW)

    seg_sum = pl.pallas_call(
        _seg_ce_kernel,
        grid=(b, nb),
        in_specs=[
            pl.BlockSpec((1, c, _SEG_BW), lambda i, j: (i, 0, j)),
            pl.BlockSpec((1, 1, 1, _SEG_BW), lambda i, j: (i, j, 0, 0)),
        ],
        out_specs=pl.BlockSpec(memory_space=pltpu.SMEM),
        out_shape=jax.ShapeDtypeStruct((1, 1), jnp.float32),
    )(logits_r, target_r)

    protos = pl.pallas_call(
        _bank_kernel,
        grid=(memory_bank.shape[0] // 128,),
        in_specs=[
            pl.BlockSpec((128, memory_bank.shape[1], _D), lambda i: (i, 0, 0)),
            pl.BlockSpec((128, _D), lambda i: (i, 0)),
        ],
        out_specs=pl.BlockSpec((128, _D), lambda i: (i, 0)),
        out_shape=jax.ShapeDtypeStruct((_C * _P, _D), jnp.float32),
    )(memory_bank, prototypes)

    rearr_emb = jnp.transpose(embedding, (0, 2, 3, 1)).reshape(-1, _D)
    n_ctr = rearr_emb.shape[0]
    contrast_lb = target[:, ::_STRIDE, ::_STRIDE].reshape(-1)
    _R = 512
    nr = n_ctr // _R
    lb_r = contrast_lb.reshape(nr, 1, _R)

    ctr_sum = pl.pallas_call(
        _contrast_kernel,
        grid=(nr,),
        in_specs=[
            pl.BlockSpec((_R, _D), lambda i: (i, 0)),
            pl.BlockSpec((_C * _P, _D), lambda i: (0, 0)),
            pl.BlockSpec((1, 1, _R), lambda i: (i, 0, 0)),
        ],
        out_specs=pl.BlockSpec(memory_space=pltpu.SMEM),
        out_shape=jax.ShapeDtypeStruct((1, 1), jnp.float32),
    )(rearr_emb, protos, lb_r)

    loss_seg = seg_sum[0, 0] / n_seg
    loss_contrast = ctr_sum[0, 0] / n_ctr
    loss = loss_seg + _LOSS_W * loss_contrast
    return (loss, loss_seg, loss_contrast, protos)


# speedup vs baseline: 1.5127x; 1.0082x over previous
import jax
import jax.numpy as jnp
from jax.experimental import pallas as pl
from jax.experimental.pallas import tpu as pltpu

_C = 128
_P = 8
_D = 256
_STRIDE = 8
_COEF = 0.999
_LOSS_W = 0.1
_EPS = 1e-12

_SEG_BW = 16384


def _seg_ce_kernel(x1_ref, x2_ref, t_ref, acc_ref):
    i = pl.program_id(0)
    j = pl.program_id(1)
    x1 = x1_ref[0]
    x2 = x2_ref[0]
    t = t_ref[0, 0, 0]
    m = jnp.maximum(jnp.max(x1, axis=0), jnp.max(x2, axis=0))
    s = (jnp.sum(jnp.exp(x1 - m[None, :]), axis=0)
         + jnp.sum(jnp.exp(x2 - m[None, :]), axis=0))
    lse = m + jnp.log(s)
    cls1 = jax.lax.broadcasted_iota(jnp.int32, x1.shape, 0)
    tsel = (jnp.sum(jnp.where(cls1 == t[None, :], x1, 0.0), axis=0)
            + jnp.sum(jnp.where(cls1 + (_C // 2) == t[None, :], x2, 0.0), axis=0))
    partial = jnp.sum(lse - tsel)

    @pl.when(jnp.logical_and(i == 0, j == 0))
    def _():
        acc_ref[0, 0] = 0.0

    acc_ref[0, 0] += partial


def _bank_kernel(mb_ref, proto_ref, out_ref):
    mean = jnp.mean(mb_ref[...], axis=1)
    n1 = jnp.sqrt(jnp.sum(mean * mean, axis=-1, keepdims=True))
    nm = mean / jnp.maximum(n1, _EPS)
    blended = nm * (1.0 - _COEF) + proto_ref[...] * _COEF
    n2 = jnp.sqrt(jnp.sum(blended * blended, axis=-1, keepdims=True))
    out_ref[...] = blended / jnp.maximum(n2, _EPS)


def _contrast_kernel(emb_ref, protos_ref, lb_ref, acc_ref):
    i = pl.program_id(0)
    logits = jax.lax.dot_general(
        emb_ref[...], protos_ref[...],
        dimension_numbers=(((1,), (1,)), ((), ())),
        preferred_element_type=jnp.float32)
    lb = lb_ref[0, 0]
    m = jnp.max(logits, axis=1)
    s = jnp.sum(jnp.exp(logits - m[:, None]), axis=1)
    lse = m + jnp.log(s)
    col = jax.lax.broadcasted_iota(jnp.int32, logits.shape, 1)
    lo = (lb * _P)[:, None]
    mask = jnp.logical_and(col >= lo, col < lo + _P)
    clsmax = jnp.max(jnp.where(mask, logits, -jnp.inf), axis=1)
    partial = jnp.sum(lse - clsmax)

    @pl.when(i == 0)
    def _():
        acc_ref[0, 0] = 0.0

    acc_ref[0, 0] += partial


def kernel(logits, embedding, memory_bank, prototypes, target, dataset_ids):
    b, c, h, w = logits.shape
    hw = h * w
    n_seg = b * hw
    nb = hw // _SEG_BW

    logits_r = logits.reshape(b, c, hw)
    target_r = target.reshape(b, nb, 1, _SEG_BW)

    seg_sum = pl.pallas_call(
        _seg_ce_kernel,
        grid=(b, nb),
        in_specs=[
            pl.BlockSpec((1, c // 2, _SEG_BW), lambda i, j: (i, 0, j)),
            pl.BlockSpec((1, c // 2, _SEG_BW), lambda i, j: (i, 1, j)),
            pl.BlockSpec((1, 1, 1, _SEG_BW), lambda i, j: (i, j, 0, 0)),
        ],
        out_specs=pl.BlockSpec(memory_space=pltpu.SMEM),
        out_shape=jax.ShapeDtypeStruct((1, 1), jnp.float32),
    )(logits_r, logits_r, target_r)

    protos = pl.pallas_call(
        _bank_kernel,
        grid=(memory_bank.shape[0] // 128,),
        in_specs=[
            pl.BlockSpec((128, memory_bank.shape[1], _D), lambda i: (i, 0, 0)),
            pl.BlockSpec((128, _D), lambda i: (i, 0)),
        ],
        out_specs=pl.BlockSpec((128, _D), lambda i: (i, 0)),
        out_shape=jax.ShapeDtypeStruct((_C * _P, _D), jnp.float32),
    )(memory_bank, prototypes)

    rearr_emb = jnp.transpose(embedding, (0, 2, 3, 1)).reshape(-1, _D)
    n_ctr = rearr_emb.shape[0]
    contrast_lb = target[:, ::_STRIDE, ::_STRIDE].reshape(-1)
    _R = 512
    nr = n_ctr // _R
    lb_r = contrast_lb.reshape(nr, 1, _R)

    ctr_sum = pl.pallas_call(
        _contrast_kernel,
        grid=(nr,),
        in_specs=[
            pl.BlockSpec((_R, _D), lambda i: (i, 0)),
            pl.BlockSpec((_C * _P, _D), lambda i: (0, 0)),
            pl.BlockSpec((1, 1, _R), lambda i: (i, 0, 0)),
        ],
        out_specs=pl.BlockSpec(memory_space=pltpu.SMEM),
        out_shape=jax.ShapeDtypeStruct((1, 1), jnp.float32),
    )(rearr_emb, protos, lb_r)

    loss_seg = seg_sum[0, 0] / n_seg
    loss_contrast = ctr_sum[0, 0] / n_ctr
    loss = loss_seg + _LOSS_W * loss_contrast
    return (loss, loss_seg, loss_contrast, protos)
